# Initial kernel scaffold; baseline (speedup 1.0000x reference)
import jax
import jax.numpy as jnp
from jax.experimental import pallas as pl


def kernel(x, pos, batch, ptr, params):
    raise NotImplementedError("write your pallas kernel here")


# speedup vs baseline: 4.4909x; 4.4909x over previous
import functools

import jax
import jax.numpy as jnp
import numpy as np
from jax import lax
from jax.experimental import pallas as pl
from jax.experimental.pallas import tpu as pltpu

_K = 16
_NS = float(1.0 / np.sqrt(1.0 + 1e-5))
_F32 = jnp.float32


def _leaky(v):
    return jnp.where(v >= 0, v, v * 0.2)


def _fold(p):
    W = p['W']
    b = p.get('b')
    if 'gamma' in p:
        g = p['gamma'] * _NS
        W = W * g[None, :]
        b = (b * g + p['beta']) if b is not None else p['beta']
    return W, (None if b is None else b.reshape(1, -1))


def _dot(a, b):
    return jnp.dot(a, b, preferred_element_type=_F32)


def _lin_body(act, x_r, w_r, b_r, o_r):
    y = _dot(x_r[...], w_r[...]) + b_r[...]
    o_r[...] = _leaky(y) if act else y


def _linear(x, W, b, act):
    return pl.pallas_call(
        functools.partial(_lin_body, act),
        out_shape=jax.ShapeDtypeStruct((x.shape[0], W.shape[1]), _F32),
    )(x, W, b)


def _pre_body(x_r, ws_r, bs_r, wm_r, bm_r, sc_r, h_r):
    xv = x_r[...]
    sc_r[...] = _dot(xv, ws_r[...]) + bs_r[...]
    h_r[...] = _leaky(_dot(xv, wm_r[...]) + bm_r[...])


def _blockpre(x, Ws, bs, Wm, bm):
    n = x.shape[0]
    return pl.pallas_call(
        _pre_body,
        out_shape=[jax.ShapeDtypeStruct((n, Ws.shape[1]), _F32),
                   jax.ShapeDtypeStruct((n, Wm.shape[1]), _F32)],
    )(x, Ws, bs, Wm, bm)


def _post_body(h_r, sc_r, w_r, b_r, o_r):
    o_r[...] = _leaky(_dot(h_r[...], w_r[...]) + b_r[...] + sc_r[...])


def _blockpost(h, sc, W, b):
    return pl.pallas_call(
        _post_body,
        out_shape=jax.ShapeDtypeStruct((h.shape[0], W.shape[1]), _F32),
    )(h, sc, W, b)


def _knn_body(n, tq, pos_r, posT_r, out_r):
    def tile(t, carry):
        base = t * tq
        q = pos_r[pl.ds(base, tq), :]
        d = None
        for c in range(3):
            df = q[:, c:c + 1] - posT_r[c:c + 1, :]
            d = df * df if d is None else d + df * df
        iota = lax.broadcasted_iota(jnp.int32, (tq, n), 1)
        for j in range(_K):
            dmin = jnp.min(d, axis=1, keepdims=True)
            idxc = jnp.min(jnp.where(d == dmin, iota, n), axis=1, keepdims=True)
            out_r[pl.ds(base, tq), pl.ds(j, 1)] = idxc
            d = jnp.where(iota == idxc, jnp.float32(jnp.inf), d)
        return carry

    lax.fori_loop(0, n // tq, tile, 0)


def _knn(pos_p, posT, n):
    tq = min(n, 512)
    return pl.pallas_call(
        functools.partial(_knn_body, n, tq),
        out_shape=jax.ShapeDtypeStruct((n, _K), jnp.int32),
    )(pos_p, posT)


def _gather_body(nb, tq, idx_r, tab_r, out_r):
    tab = tab_r[...]
    iota = lax.broadcasted_iota(jnp.int32, (tq, nb), 1)
    for j in range(_K):
        idq = idx_r[:, pl.ds(j, 1)]
        oh = (iota == idq).astype(_F32)
        out_r[j, :, :] = _dot(oh, tab)


def _gather_rows(nbr, tab):
    n = nbr.shape[0]
    nb, Dp = tab.shape
    tq = min(n, 512)
    return pl.pallas_call(
        functools.partial(_gather_body, nb, tq),
        grid=(n // tq,),
        in_specs=[pl.BlockSpec((tq, _K), lambda t: (t, 0)),
                  pl.BlockSpec((nb, Dp), lambda t: (0, 0))],
        out_specs=pl.BlockSpec((_K, tq, Dp), lambda t: (0, t, 0)),
        out_shape=jax.ShapeDtypeStruct((_K, n, Dp), _F32),
    )(nbr, tab)


def _mk_tab(posn, h):
    D = 3 + h.shape[1]
    Dp = ((D + 15) // 16) * 16
    pad = jnp.zeros((h.shape[0], Dp - D), _F32)
    return jnp.concatenate([posn, h, pad], axis=1)


def _lfa_body(ce, g_r, pos_r, wpi_r, wpj_r, wdf_r, wds_r, be_r,
              axx_r, axe_r, aex_r, aee_r, px_r, pe_r, bp_r, out_r):
    pos_i = pos_r[:, 0:3]
    pit = _dot(pos_i, wpi_r[...])
    encs, xjs, axs, aes = [], [], [], []
    mx = me = None
    for j in range(_K):
        pj = g_r[j, :, 0:3]
        xj = g_r[j, :, 3:3 + ce]
        df = pj - pos_i
        dist = jnp.sqrt(jnp.sum(df * df, axis=1, keepdims=True))
        enc = _leaky(pit + _dot(pj, wpj_r[...]) + _dot(df, wdf_r[...])
                     + dist * wds_r[...] + be_r[...])
        ax = _dot(xj, axx_r[...]) + _dot(enc, aex_r[...])
        ae = _dot(xj, axe_r[...]) + _dot(enc, aee_r[...])
        encs.append(enc)
        xjs.append(xj)
        axs.append(ax)
        aes.append(ae)
        mx = ax if mx is None else jnp.maximum(mx, ax)
        me = ae if me is None else jnp.maximum(me, ae)
    sx = se = aggx = agge = None
    for j in range(_K):
        ex = jnp.exp(axs[j] - mx)
        ee = jnp.exp(aes[j] - me)
        px_ = ex * xjs[j]
        pe_ = ee * encs[j]
        if j == 0:
            sx, se, aggx, agge = ex, ee, px_, pe_
        else:
            sx += ex
            se += ee
            aggx += px_
            agge += pe_
    out_r[...] = _leaky(_dot(aggx / sx, px_r[...]) + _dot(agge / se, pe_r[...])
                        + bp_r[...])


def _lfa(g, pos_p, lp, ce, n):
    We, be = _fold(lp['enc'][0])
    Wa, _ = _fold(lp['att'][0])
    Wp, bp = _fold(lp['post'][0])
    wpi, wpj, wdf, wds = We[0:3], We[3:6], We[6:9], We[9:10]
    axx, axe = Wa[:ce, :ce], Wa[:ce, ce:]
    aex, aee = Wa[ce:, :ce], Wa[ce:, ce:]
    px, pe = Wp[:ce], Wp[ce:]
    Dp = g.shape[2]
    tq = min(n, 512)
    _w = lambda a: pl.BlockSpec(a.shape, lambda t: (0,) * a.ndim)
    return pl.pallas_call(
        functools.partial(_lfa_body, ce),
        grid=(n // tq,),
        in_specs=[pl.BlockSpec((_K, tq, Dp), lambda t: (0, t, 0)),
                  pl.BlockSpec((tq, 8), lambda t: (t, 0)),
                  _w(wpi), _w(wpj), _w(wdf), _w(wds), _w(be),
                  _w(axx), _w(axe), _w(aex), _w(aee), _w(px), _w(pe), _w(bp)],
        out_specs=pl.BlockSpec((tq, 2 * ce), lambda t: (t, 0)),
        out_shape=jax.ShapeDtypeStruct((n, 2 * ce), _F32),
    )(g, pos_p, wpi, wpj, wdf, wds, be, axx, axe, aex, aee, px, pe, bp)


def _itp_body(nq, nb, tq, q_r, bT_r, f_r, sk_r, w1_r, w2_r, b_r, out_r):
    feats = f_r[...]

    def tile(t, carry):
        base = t * tq
        q = q_r[pl.ds(base, tq), :]
        d = None
        for c in range(3):
            df = q[:, c:c + 1] - bT_r[c:c + 1, :]
            d = df * df if d is None else d + df * df
        iota = lax.broadcasted_iota(jnp.int32, (tq, nb), 1)
        dmin = jnp.min(d, axis=1, keepdims=True)
        idxc = jnp.min(jnp.where(d == dmin, iota, nb), axis=1, keepdims=True)
        oh = (iota == idxc).astype(_F32)
        interp = _dot(oh, feats)
        y = _dot(interp, w1_r[...]) + _dot(sk_r[pl.ds(base, tq), :], w2_r[...]) + b_r[...]
        out_r[pl.ds(base, tq), :] = _leaky(y)
        return carry

    lax.fori_loop(0, nq // tq, tile, 0)


def _interp_fp(posq, posbT, feats, skip, W1, W2, b, nq, nb):
    tq = min(nq, 512)
    return pl.pallas_call(
        functools.partial(_itp_body, nq, nb, tq),
        out_shape=jax.ShapeDtypeStruct((nq, W1.shape[1]), _F32),
    )(posq, posbT, feats, skip, W1, W2, b)


def _end_body(x_r, w1_r, b1_r, w2_r, b2_r, w3_r, b3_r, out_r):
    h = _leaky(_dot(x_r[...], w1_r[...]) + b1_r[...])
    h = _leaky(_dot(h, w2_r[...]) + b2_r[...])
    lg = _dot(h, w3_r[...]) + b3_r[...]
    sh = lg - jnp.max(lg, axis=1, keepdims=True)
    out_r[...] = sh - jnp.log(jnp.sum(jnp.exp(sh), axis=1, keepdims=True))


def _end(f1, W1, b1, W2, b2, W3, b3):
    return pl.pallas_call(
        _end_body,
        out_shape=jax.ShapeDtypeStruct((f1.shape[0], W3.shape[1]), _F32),
    )(f1, W1, b1, W2, b2, W3, b3)


def kernel(x, pos, batch, ptr, params):
    P = params
    pos_p = jnp.pad(pos, ((0, 0), (0, 5)))
    posT = jnp.pad(pos.T, ((0, 5), (0, 0)))

    Wf, bf = _fold(P['fc0'])
    x0 = _linear(x, Wf, bf, act=False)

    def block(bp, xin, n, dout):
        posn = pos_p[:n]
        posTn = posT[:, :n]
        nbr = _knn(posn, posTn, n)
        Ws, bs = _fold(bp['shortcut'][0])
        Wm, bm = _fold(bp['mlp1'][0])
        sc, h = _blockpre(xin, Ws, bs, Wm, bm)
        ce1 = dout // 8
        g1 = _gather_rows(nbr, _mk_tab(pos[:n], h))
        h1 = _lfa(g1, posn, bp['lfa1'], ce1, n)
        ce2 = dout // 4
        g2 = _gather_rows(nbr, _mk_tab(pos[:n], h1))
        h2 = _lfa(g2, posn, bp['lfa2'], ce2, n)
        W2, b2 = _fold(bp['mlp2'][0])
        return _blockpost(h2, sc, W2, b2)

    x1 = block(P['b1'], x0, 4096, 32)
    x2 = block(P['b2'], x1[:1024], 1024, 128)
    x3 = block(P['b3'], x2[:256], 256, 256)
    x4 = block(P['b4'], x3[:64], 64, 512)

    Wsm, bsm = _fold(P['summit'][0])
    xs = _linear(x4[:16], Wsm, bsm, act=True)

    def fp(mp, fe, nq, nb, skip):
        W, b = _fold(mp[0])
        cf = fe.shape[1]
        return _interp_fp(pos_p[:nq], posT[:, :nb], fe, skip, W[:cf], W[cf:], b, nq, nb)

    f4 = fp(P['fp4'], xs, 64, 16, x3[:64])
    f3 = fp(P['fp3'], f4, 256, 64, x2[:256])
    f2 = fp(P['fp2'], f3, 1024, 256, x1[:1024])
    f1 = fp(P['fp1'], f2, 4096, 1024, x1)

    W1, b1 = _fold(P['end_mlp'][0])
    W2, b2 = _fold(P['end_mlp'][1])
    W3, b3 = _fold(P['end_lin'])
    return _end(f1, W1, b1, W2, b2, W3, b3)


# speedup vs baseline: 5.5479x; 1.2354x over previous
import functools

import jax
import jax.numpy as jnp
import numpy as np
from jax import lax
from jax.experimental import pallas as pl
from jax.experimental.pallas import tpu as pltpu
from jax.experimental.pallas import tpu_sc as plsc

_K = 16
_NS = float(1.0 / np.sqrt(1.0 + 1e-5))
_F32 = jnp.float32


def _leaky(v):
    return jnp.where(v >= 0, v, v * 0.2)


def _fold(p):
    W = p['W']
    b = p.get('b')
    if 'gamma' in p:
        g = p['gamma'] * _NS
        W = W * g[None, :]
        b = (b * g + p['beta']) if b is not None else p['beta']
    return W, (None if b is None else b.reshape(1, -1))


def _dot(a, b):
    return jnp.dot(a, b, preferred_element_type=_F32)


def _lin_body(act, x_r, w_r, b_r, o_r):
    y = _dot(x_r[...], w_r[...]) + b_r[...]
    o_r[...] = _leaky(y) if act else y


def _linear(x, W, b, act):
    return pl.pallas_call(
        functools.partial(_lin_body, act),
        out_shape=jax.ShapeDtypeStruct((x.shape[0], W.shape[1]), _F32),
    )(x, W, b)


def _pre_body(x_r, ws_r, bs_r, wm_r, bm_r, sc_r, h_r):
    xv = x_r[...]
    sc_r[...] = _dot(xv, ws_r[...]) + bs_r[...]
    h_r[...] = _leaky(_dot(xv, wm_r[...]) + bm_r[...])


def _blockpre(x, Ws, bs, Wm, bm):
    n = x.shape[0]
    return pl.pallas_call(
        _pre_body,
        out_shape=[jax.ShapeDtypeStruct((n, Ws.shape[1]), _F32),
                   jax.ShapeDtypeStruct((n, Wm.shape[1]), _F32)],
    )(x, Ws, bs, Wm, bm)


def _post_body(h_r, sc_r, w_r, b_r, o_r):
    o_r[...] = _leaky(_dot(h_r[...], w_r[...]) + b_r[...] + sc_r[...])


def _blockpost(h, sc, W, b):
    return pl.pallas_call(
        _post_body,
        out_shape=jax.ShapeDtypeStruct((h.shape[0], W.shape[1]), _F32),
    )(h, sc, W, b)


def _knn_body(n, tq, pos_r, posT_r, out_r):
    iota = lax.broadcasted_iota(jnp.int32, (tq, n), 1)

    def tile(t, carry):
        base = t * tq
        q = pos_r[pl.ds(base, tq), :]
        d = None
        for c in range(3):
            df = q[:, c:c + 1] - posT_r[c:c + 1, :]
            d = df * df if d is None else d + df * df
        for j in range(_K):
            dmin = jnp.min(d, axis=1, keepdims=True)
            eqm = d == dmin
            idxc = jnp.min(jnp.where(eqm, iota, n), axis=1, keepdims=True)
            out_r[pl.ds(base, tq), pl.ds(j, 1)] = idxc
            d = jnp.where(eqm, jnp.float32(jnp.inf), d)
        return carry

    lax.fori_loop(0, n // tq, tile, 0)


def _knn(pos_p, posT, n):
    tq = min(n, 512)
    return pl.pallas_call(
        functools.partial(_knn_body, n, tq),
        out_shape=jax.ShapeDtypeStruct((n, _K), jnp.int32),
    )(pos_p, posT)


def _gather_rows(nbr, tab):
    n = nbr.shape[0]
    nb, Dp = tab.shape
    B = _K * n
    info = plsc.get_sparse_core_info()
    NC, NS = info.num_cores, info.num_subcores
    NW = NC * NS
    bpw = B // NW
    ch = bpw if bpw <= 128 else 128
    nch = bpw // ch
    idx3 = nbr.T.reshape(NW, nch, ch)
    mesh = plsc.VectorSubcoreMesh(core_axis_name="c", subcore_axis_name="s")

    @functools.partial(
        pl.kernel, mesh=mesh,
        out_type=jax.ShapeDtypeStruct((B, Dp), _F32),
        scratch_types=[
            pltpu.VMEM((nch, ch), jnp.int32),
            pltpu.VMEM((2, ch, Dp), _F32),
            pltpu.SemaphoreType.DMA,
        ],
    )
    def k(tab_hbm, idx_hbm, out_hbm, idx_v, buf_v, sem):
        wid = lax.axis_index("s") * NC + lax.axis_index("c")
        base = wid * bpw
        pltpu.sync_copy(idx_hbm.at[wid], idx_v)
        descs = [pltpu.async_copy(tab_hbm.at[idx_v.at[0]], buf_v.at[0], sem)]
        for c in range(nch):
            if c + 1 < nch:
                descs.append(pltpu.async_copy(
                    tab_hbm.at[idx_v.at[c + 1]], buf_v.at[(c + 1) % 2], sem))
            descs[c].wait()
            pltpu.sync_copy(buf_v.at[c % 2],
                            out_hbm.at[pl.ds(base + c * ch, ch)])

    return k(tab, idx3).reshape(_K, n, Dp)


def _mk_tab(posn, h):
    D = h.shape[1] + 3
    Dp = ((D + 127) // 128) * 128
    pad = jnp.zeros((h.shape[0], Dp - D), _F32)
    return jnp.concatenate([h, posn, pad], axis=1)


def _lfa_body(ce, g_r, pos_r, wpi_r, wpj_r, wds_r, be_r,
              axx_r, axe_r, aex_r, aee_r, px_r, pe_r, bp_r, out_r):
    pos_i = pos_r[:, 0:3]
    wpj = wpj_r[...]
    wds = wds_r[...]
    axx, axe, aex, aee = axx_r[...], axe_r[...], aex_r[...], aee_r[...]
    pit = _dot(pos_i, wpi_r[...]) + be_r[...]
    encs, xjs, axs, aes = [], [], [], []
    mx = me = None
    for j in range(_K):
        xj = g_r[j, :, 0:ce]
        pj = g_r[j, :, ce:ce + 3]
        df = pj - pos_i
        dist = jnp.sqrt(jnp.sum(df * df, axis=1, keepdims=True))
        enc = _leaky(pit + _dot(pj, wpj) + dist * wds)
        ax = _dot(xj, axx) + _dot(enc, aex)
        ae = _dot(xj, axe) + _dot(enc, aee)
        encs.append(enc)
        xjs.append(xj)
        axs.append(ax)
        aes.append(ae)
        mx = ax if mx is None else jnp.maximum(mx, ax)
        me = ae if me is None else jnp.maximum(me, ae)
    sx = se = aggx = agge = None
    for j in range(_K):
        ex = jnp.exp(axs[j] - mx)
        ee = jnp.exp(aes[j] - me)
        px_ = ex * xjs[j]
        pe_ = ee * encs[j]
        if j == 0:
            sx, se, aggx, agge = ex, ee, px_, pe_
        else:
            sx += ex
            se += ee
            aggx += px_
            agge += pe_
    out_r[...] = _leaky(_dot(aggx / sx, px_r[...]) + _dot(agge / se, pe_r[...])
                        + bp_r[...])


def _lfa(g, pos_p, lp, ce, n):
    We, be = _fold(lp['enc'][0])
    Wa, _ = _fold(lp['att'][0])
    Wp, bp = _fold(lp['post'][0])
    wpi, wpj, wdf, wds = We[0:3], We[3:6], We[6:9], We[9:10]
    wpi = wpi - wdf
    wpj = wpj + wdf
    axx, axe = Wa[:ce, :ce], Wa[:ce, ce:]
    aex, aee = Wa[ce:, :ce], Wa[ce:, ce:]
    px, pe = Wp[:ce], Wp[ce:]
    Dp = g.shape[2]
    tq = min(n, 512)
    _w = lambda a: pl.BlockSpec(a.shape, lambda t: (0,) * a.ndim)
    return pl.pallas_call(
        functools.partial(_lfa_body, ce),
        grid=(n // tq,),
        in_specs=[pl.BlockSpec((_K, tq, Dp), lambda t: (0, t, 0)),
                  pl.BlockSpec((tq, 8), lambda t: (t, 0)),
                  _w(wpi), _w(wpj), _w(wds), _w(be),
                  _w(axx), _w(axe), _w(aex), _w(aee), _w(px), _w(pe), _w(bp)],
        out_specs=pl.BlockSpec((tq, 2 * ce), lambda t: (t, 0)),
        out_shape=jax.ShapeDtypeStruct((n, 2 * ce), _F32),
    )(g, pos_p, wpi, wpj, wds, be, axx, axe, aex, aee, px, pe, bp)


def _itp_body(nq, nb, tq, q_r, bT_r, f_r, sk_r, w1_r, w2_r, b_r, out_r):
    feats = f_r[...]
    iota = lax.broadcasted_iota(jnp.int32, (tq, nb), 1)

    def tile(t, carry):
        base = t * tq
        q = q_r[pl.ds(base, tq), :]
        d = None
        for c in range(3):
            df = q[:, c:c + 1] - bT_r[c:c + 1, :]
            d = df * df if d is None else d + df * df
        dmin = jnp.min(d, axis=1, keepdims=True)
        idxc = jnp.min(jnp.where(d == dmin, iota, nb), axis=1, keepdims=True)
        oh = (iota == idxc).astype(_F32)
        interp = _dot(oh, feats)
        y = _dot(interp, w1_r[...]) + _dot(sk_r[pl.ds(base, tq), :], w2_r[...]) + b_r[...]
        out_r[pl.ds(base, tq), :] = _leaky(y)
        return carry

    lax.fori_loop(0, nq // tq, tile, 0)


def _interp_fp(posq, posbT, feats, skip, W1, W2, b, nq, nb):
    tq = min(nq, 512)
    return pl.pallas_call(
        functools.partial(_itp_body, nq, nb, tq),
        out_shape=jax.ShapeDtypeStruct((nq, W1.shape[1]), _F32),
    )(posq, posbT, feats, skip, W1, W2, b)


def _knn_oh_val(pos3, posT, n):
    d = None
    for c in range(3):
        df = pos3[:, c:c + 1] - posT[c:c + 1, :]
        d = df * df if d is None else d + df * df
    iota = lax.broadcasted_iota(jnp.int32, (n, n), 1)
    ohs = []
    for j in range(_K):
        dmin = jnp.min(d, axis=1, keepdims=True)
        eqm = d == dmin
        idxc = jnp.min(jnp.where(eqm, iota, n), axis=1, keepdims=True)
        ohs.append((iota == idxc).astype(_F32))
        d = jnp.where(eqm, jnp.float32(jnp.inf), d)
    return ohs


def _lfa_val(ohs, h, pos3, w):
    wpi, wpj, wds, be, axx, axe, aex, aee, px, pe, bp = w
    pit = _dot(pos3, wpi) + be
    xjs, encs, axs, aes = [], [], [], []
    mx = me = None
    for oh in ohs:
        xj = _dot(oh, h)
        pj = _dot(oh, pos3)
        df = pj - pos3
        dist = jnp.sqrt(jnp.sum(df * df, axis=1, keepdims=True))
        enc = _leaky(pit + _dot(pj, wpj) + dist * wds)
        ax = _dot(xj, axx) + _dot(enc, aex)
        ae = _dot(xj, axe) + _dot(enc, aee)
        xjs.append(xj)
        encs.append(enc)
        axs.append(ax)
        aes.append(ae)
        mx = ax if mx is None else jnp.maximum(mx, ax)
        me = ae if me is None else jnp.maximum(me, ae)
    sx = se = aggx = agge = None
    for j in range(_K):
        ex = jnp.exp(axs[j] - mx)
        ee = jnp.exp(aes[j] - me)
        px_ = ex * xjs[j]
        pe_ = ee * encs[j]
        if j == 0:
            sx, se, aggx, agge = ex, ee, px_, pe_
        else:
            sx += ex
            se += ee
            aggx += px_
            agge += pe_
    return _leaky(_dot(aggx / sx, px) + _dot(agge / se, pe) + bp)


def _itp_val(q3, bT, feats, skip, w1, w2, b):
    nq = q3.shape[0]
    nb = bT.shape[1]
    d = None
    for c in range(3):
        df = q3[:, c:c + 1] - bT[c:c + 1, :]
        d = df * df if d is None else d + df * df
    iota = lax.broadcasted_iota(jnp.int32, (nq, nb), 1)
    dmin = jnp.min(d, axis=1, keepdims=True)
    idxc = jnp.min(jnp.where(d == dmin, iota, nb), axis=1, keepdims=True)
    oh = (iota == idxc).astype(_F32)
    return _leaky(_dot(_dot(oh, feats), w1) + _dot(skip, w2) + b)


def _lfa_wlist(lp, ce):
    We, be = _fold(lp['enc'][0])
    Wa, _ = _fold(lp['att'][0])
    Wp, bp = _fold(lp['post'][0])
    wpi, wpj, wdf, wds = We[0:3], We[3:6], We[6:9], We[9:10]
    return [wpi - wdf, wpj + wdf, wds, be,
            Wa[:ce, :ce], Wa[:ce, ce:], Wa[ce:, :ce], Wa[ce:, ce:],
            Wp[:ce], Wp[ce:], bp]


def _tail_body(x_r, pos_r, posT_r, *rest):
    out_r = rest[-1]
    it = iter(rest[:-1])
    nxt = lambda: next(it)[...]

    pos3 = pos_r[:, 0:3]
    posT = posT_r[...]

    def blkv(x, n, dout):
        p3 = pos3[:n]
        ohs = _knn_oh_val(p3, posT[:, :n], n)
        Ws, bs, Wm, bm = nxt(), nxt(), nxt(), nxt()
        sc = _dot(x, Ws) + bs
        h = _leaky(_dot(x, Wm) + bm)
        h = _lfa_val(ohs, h, p3, [nxt() for _ in range(11)])
        h = _lfa_val(ohs, h, p3, [nxt() for _ in range(11)])
        Wm2, bm2 = nxt(), nxt()
        return _leaky(_dot(h, Wm2) + bm2 + sc)

    x2d = x_r[...]
    x3 = blkv(x2d, 256, 256)
    x4 = blkv(x3[:64], 64, 512)
    xs = _leaky(_dot(x4[:16], nxt()) + nxt())
    f4 = _itp_val(pos3[:64], posT[:, :16], xs, x3[:64], nxt(), nxt(), nxt())
    f3 = _itp_val(pos3, posT[:, :64], f4, x2d, nxt(), nxt(), nxt())
    out_r[...] = f3


def _tail(x2d, pos_p, posT, P):
    def lin(p):
        W, b = _fold(p)
        return [W, b]

    def blk(bp, dout):
        return (lin(bp['shortcut'][0]) + lin(bp['mlp1'][0])
                + _lfa_wlist(bp['lfa1'], dout // 8)
                + _lfa_wlist(bp['lfa2'], dout // 4)
                + lin(bp['mlp2'][0]))

    def fpw(mp, cf):
        W, b = _fold(mp[0])
        return [W[:cf], W[cf:], b]

    arrs = (blk(P['b3'], 256) + blk(P['b4'], 512) + lin(P['summit'][0])
            + fpw(P['fp4'], 512) + fpw(P['fp3'], 256))
    return pl.pallas_call(
        _tail_body,
        out_shape=jax.ShapeDtypeStruct((256, 128), _F32),
    )(x2d, pos_p[:256], posT[:, :256], *arrs)


def _end_body(x_r, w1_r, b1_r, w2_r, b2_r, w3_r, b3_r, out_r):
    h = _leaky(_dot(x_r[...], w1_r[...]) + b1_r[...])
    h = _leaky(_dot(h, w2_r[...]) + b2_r[...])
    lg = _dot(h, w3_r[...]) + b3_r[...]
    sh = lg - jnp.max(lg, axis=1, keepdims=True)
    out_r[...] = sh - jnp.log(jnp.sum(jnp.exp(sh), axis=1, keepdims=True))


def _end(f1, W1, b1, W2, b2, W3, b3):
    return pl.pallas_call(
        _end_body,
        out_shape=jax.ShapeDtypeStruct((f1.shape[0], W3.shape[1]), _F32),
    )(f1, W1, b1, W2, b2, W3, b3)


def kernel(x, pos, batch, ptr, params):
    P = params
    pos_p = jnp.pad(pos, ((0, 0), (0, 5)))
    posT = jnp.pad(pos.T, ((0, 5), (0, 0)))

    Wf, bf = _fold(P['fc0'])
    x0 = _linear(x, Wf, bf, act=False)

    def block(bp, xin, n, dout):
        posn = pos_p[:n]
        posTn = posT[:, :n]
        nbr = _knn(posn, posTn, n)
        Ws, bs = _fold(bp['shortcut'][0])
        Wm, bm = _fold(bp['mlp1'][0])
        sc, h = _blockpre(xin, Ws, bs, Wm, bm)
        ce1 = dout // 8
        g1 = _gather_rows(nbr, _mk_tab(pos[:n], h))
        h1 = _lfa(g1, posn, bp['lfa1'], ce1, n)
        ce2 = dout // 4
        g2 = _gather_rows(nbr, _mk_tab(pos[:n], h1))
        h2 = _lfa(g2, posn, bp['lfa2'], ce2, n)
        W2, b2 = _fold(bp['mlp2'][0])
        return _blockpost(h2, sc, W2, b2)

    x1 = block(P['b1'], x0, 4096, 32)
    x2 = block(P['b2'], x1[:1024], 1024, 128)
    f3 = _tail(x2[:256], pos_p, posT, P)

    def fp(mp, fe, nq, nb, skip):
        W, b = _fold(mp[0])
        cf = fe.shape[1]
        return _interp_fp(pos_p[:nq], posT[:, :nb], fe, skip, W[:cf], W[cf:], b, nq, nb)

    f2 = fp(P['fp2'], f3, 1024, 256, x1[:1024])
    f1 = fp(P['fp1'], f2, 4096, 1024, x1)

    W1, b1 = _fold(P['end_mlp'][0])
    W2, b2 = _fold(P['end_mlp'][1])
    W3, b3 = _fold(P['end_lin'])
    return _end(f1, W1, b1, W2, b2, W3, b3)
